# baseline (speedup 1.0000x reference)
import functools

import jax
import jax.numpy as jnp
from jax import lax
from jax.experimental import pallas as pl
from jax.experimental.pallas import tpu as pltpu
from jax.experimental.pallas import tpu_sc as plsc

_NUM_CORES = 2
_NUM_SUBCORES = 16
_NUM_WORKERS = _NUM_CORES * _NUM_SUBCORES
_LANES = 128
_CHUNK = 1024
_NCHUNK = 32
_HIT_CAP = 4096
_CHIT_CAP = 512
_RING = 16


def _make_gather(B, D, T):
    ncols = (T + _LANES - 1) // _LANES
    mesh = plsc.VectorSubcoreMesh(core_axis_name="c", subcore_axis_name="s")

    @functools.partial(
        pl.kernel,
        mesh=mesh,
        out_type=(
            jax.ShapeDtypeStruct((B * D + _NUM_WORKERS * D,), jnp.float32),
            jax.ShapeDtypeStruct((B * D + _NUM_WORKERS * D,), jnp.float32),
        ),
        scratch_types=[
            pltpu.VMEM((B,), jnp.int32),
            pltpu.VMEM((2, D, _CHUNK), jnp.float32),
            pltpu.VMEM((_HIT_CAP + 16,), jnp.int32),
            pltpu.VMEM((_HIT_CAP + 16,), jnp.int32),
            pltpu.VMEM((_CHIT_CAP + 32,), jnp.int32),
            pltpu.VMEM((_CHIT_CAP + 32,), jnp.int32),
            pltpu.VMEM((_RING, D), jnp.float32),
            pltpu.SemaphoreType.DMA((2,)),
            pltpu.SemaphoreType.DMA((_RING,)),
        ],
        compiler_params=pltpu.CompilerParams(use_tc_tiling_on_sc=True,
                                             needs_layout_passes=False),
    )
    def gather_kernel(uidx_hbm, iidx_hbm, utT_hbm, itT_hbm,
                      uout_hbm, iout_hbm,
                      idx_v, stage_v, hi_v, hb_v, ci_v, cb_v, ring_v,
                      ssem, rsem):
        wid = lax.axis_index("s") * _NUM_CORES + lax.axis_index("c")
        slab_lo = wid * (_NCHUNK * _CHUNK)
        iota = lax.iota(jnp.int32, 16)
        jrow = iota
        jhi = iota + 16

        def chunk_col(c):
            nom = wid * (_NCHUNK * _CHUNK // _LANES) + c * (_CHUNK // _LANES)
            return jnp.minimum(nom, ncols - _CHUNK // _LANES)

        def issue_chunk(tabT_hbm, c, par):
            start = pl.multiple_of(chunk_col(c) * _LANES, _LANES)
            pltpu.async_copy(tabT_hbm.at[:, pl.ds(start, _CHUNK)],
                             stage_v.at[par], ssem.at[par])

        def wait_chunk(tabT_hbm, par):
            pltpu.make_async_copy(tabT_hbm.at[:, pl.ds(0, _CHUNK)],
                                  stage_v.at[par], ssem.at[par]).wait()

        def wait_ring(out_hbm, l):
            pltpu.make_async_copy(out_hbm.at[pl.ds(0, D)],
                                  ring_v.at[l], rsem.at[l]).wait()

        def run_table(idx_hbm, tabT_hbm, out_hbm):
            pltpu.sync_copy(idx_hbm, idx_v)

            def scan(k, off):
                v = idx_v[pl.ds(k * 16, 16)]
                m = (v >= slab_lo) & (v < slab_lo + _NCHUNK * _CHUNK)
                b = k * 16 + iota
                offc = jnp.minimum(off, _HIT_CAP)
                plsc.store_compressed(hi_v.at[pl.ds(offc, 16)], v, mask=m)
                plsc.store_compressed(hb_v.at[pl.ds(offc, 16)], b, mask=m)
                return off + plsc.all_reduce_population_count(m)[0]

            cnt = jnp.minimum(lax.fori_loop(0, B // 16, scan, 0), _HIT_CAP)

            pad_off = B * D + wid * D
            for l in range(_RING):
                pltpu.async_copy(ring_v.at[l], out_hbm.at[pl.ds(pad_off, D)],
                                 rsem.at[l])
            issue_chunk(tabT_hbm, 0, 0)
            issue_chunk(tabT_hbm, 1, 1)

            def do_chunk(c, par):
                wait_chunk(tabT_hbm, par)
                base = chunk_col(c) * _LANES
                nom_lo = slab_lo + c * _CHUNK

                def cscan(k, off):
                    v = hi_v[pl.ds(k * 16, 16)]
                    b = hb_v[pl.ds(k * 16, 16)]
                    m = (v >= nom_lo) & (v < nom_lo + _CHUNK)
                    offc = jnp.minimum(off, _CHIT_CAP)
                    plsc.store_compressed(ci_v.at[pl.ds(offc, 16)], v, mask=m)
                    plsc.store_compressed(cb_v.at[pl.ds(offc, 16)], b, mask=m)
                    return off + plsc.all_reduce_population_count(m)[0]

                nvec = (cnt + 15) >> 4
                ccnt = jnp.minimum(lax.fori_loop(0, nvec, cscan, 0),
                                   _CHIT_CAP)
                bfill = jnp.broadcast_to(B + wid, (16,))
                ci_v[pl.ds(ccnt, 16)] = jnp.broadcast_to(base, (16,))
                cb_v[pl.ds(ccnt, 16)] = bfill
                ci_v[pl.ds(ccnt + 16, 16)] = jnp.broadcast_to(base, (16,))
                cb_v[pl.ds(ccnt + 16, 16)] = bfill

                def group(g, _):
                    for half in range(2):
                        civ = ci_v[pl.ds(g * 32 + half * 16, 16)] - base
                        cbv = cb_v[pl.ds(g * 32 + half * 16, 16)]
                        for l in range(16):
                            slot = (half * 16 + l) % _RING
                            wait_ring(out_hbm, slot)
                            col = jnp.broadcast_to(civ[l], (16,))
                            g0 = plsc.load_gather(stage_v.at[par],
                                                  [jrow, col])
                            g1 = plsc.load_gather(stage_v.at[par],
                                                  [jhi, col])
                            ring_v[slot, pl.ds(0, 16)] = g0
                            ring_v[slot, pl.ds(16, 16)] = g1
                            b = cbv[l]
                            pltpu.async_copy(
                                ring_v.at[slot], out_hbm.at[pl.ds(b * D, D)],
                                rsem.at[slot])
                    return _

                lax.fori_loop(0, (ccnt + 32) >> 5, group, 0)
                issue_chunk(tabT_hbm, c + 2, par)

            def pair(cp, _):
                do_chunk(cp * 2, 0)
                do_chunk(cp * 2 + 1, 1)
                return _

            lax.fori_loop(0, _NCHUNK // 2, pair, 0)
            wait_chunk(tabT_hbm, 0)
            wait_chunk(tabT_hbm, 1)
            for l in range(_RING):
                wait_ring(out_hbm, l)

        run_table(uidx_hbm, utT_hbm, uout_hbm)
        run_table(iidx_hbm, itT_hbm, iout_hbm)

    return gather_kernel


def _mlp_body(u_ref, v_ref, w1a_ref, w1b_ref, b1_ref, w2_ref, b2_ref,
              w3_ref, b3_ref, w4_ref, b4_ref, out_ref):
    dot = functools.partial(jnp.dot, preferred_element_type=jnp.float32,
                            precision=lax.Precision.HIGHEST)
    x = dot(u_ref[...], w1a_ref[...]) + dot(v_ref[...], w1b_ref[...])
    x = jnp.maximum(x + b1_ref[...], 0.0)
    x = jnp.maximum(dot(x, w2_ref[...]) + b2_ref[...], 0.0)
    x = jnp.maximum(dot(x, w3_ref[...]) + b3_ref[...], 0.0)
    logits = jnp.sum(x * w4_ref[...], axis=1) + b4_ref[0]
    out_ref[...] = jax.nn.sigmoid(logits)


def _make_mlp(B, D, blk):
    grid = (B // blk,)
    full = lambda shape: pl.BlockSpec(shape, lambda i: (0,) * len(shape))
    return pl.pallas_call(
        _mlp_body,
        grid=grid,
        in_specs=[
            pl.BlockSpec((blk, D), lambda i: (i, 0)),
            pl.BlockSpec((blk, D), lambda i: (i, 0)),
            full((D, 64)), full((D, 64)), full((1, 64)),
            full((64, 32)), full((1, 32)),
            full((32, 16)), full((1, 16)),
            full((1, 16)), full((1,)),
        ],
        out_specs=pl.BlockSpec((blk,), lambda i: (i,)),
        out_shape=jax.ShapeDtypeStruct((B,), jnp.float32),
    )


def kernel(user_indices, item_indices, user_table, item_table,
           W1, b1, W2, b2, W3, b3, W4, b4):
    B = user_indices.shape[0]
    D = user_table.shape[1]
    T = user_table.shape[0]
    uflat, iflat = _make_gather(B, D, T)(
        user_indices, item_indices, user_table.T, item_table.T)
    user_vec = uflat[:B * D].reshape(B, D)
    item_vec = iflat[:B * D].reshape(B, D)
    w1a = W1[:, :D].T
    w1b = W1[:, D:].T
    mlp = _make_mlp(B, D, 2048)
    return mlp(user_vec, item_vec,
               w1a, w1b, b1.reshape(1, 64),
               W2.T, b2.reshape(1, 32),
               W3.T, b3.reshape(1, 16),
               W4.reshape(1, 16), b4)


# speedup vs baseline: 1.0044x; 1.0044x over previous
import functools

import jax
import jax.numpy as jnp
from jax import lax
from jax.experimental import pallas as pl
from jax.experimental.pallas import tpu as pltpu
from jax.experimental.pallas import tpu_sc as plsc

_NUM_CORES = 2
_NUM_SUBCORES = 16
_NUM_WORKERS = _NUM_CORES * _NUM_SUBCORES
_LANES = 128
_CHUNK = 1024
_NCHUNK = 32
_HIT_CAP = 4096
_CHIT_CAP = 512
_RING = 16


def _make_gather(B, D, T):
    ncols = (T + _LANES - 1) // _LANES
    mesh = plsc.VectorSubcoreMesh(core_axis_name="c", subcore_axis_name="s")

    @functools.partial(
        pl.kernel,
        mesh=mesh,
        out_type=(
            jax.ShapeDtypeStruct((B * D + _NUM_WORKERS * D,), jnp.float32),
            jax.ShapeDtypeStruct((B * D + _NUM_WORKERS * D,), jnp.float32),
        ),
        scratch_types=[
            pltpu.VMEM((B,), jnp.int32),
            pltpu.VMEM((2, D, _CHUNK), jnp.float32),
            pltpu.VMEM((_HIT_CAP + 16,), jnp.int32),
            pltpu.VMEM((_HIT_CAP + 16,), jnp.int32),
            pltpu.VMEM((_CHIT_CAP + 32,), jnp.int32),
            pltpu.VMEM((_CHIT_CAP + 32,), jnp.int32),
            pltpu.VMEM((_RING, D), jnp.float32),
            pltpu.SemaphoreType.DMA((2,)),
            pltpu.SemaphoreType.DMA((_RING,)),
        ],
        compiler_params=pltpu.CompilerParams(use_tc_tiling_on_sc=True,
                                             needs_layout_passes=False),
    )
    def gather_kernel(uidx_hbm, iidx_hbm, utT_hbm, itT_hbm,
                      uout_hbm, iout_hbm,
                      idx_v, stage_v, hi_v, hb_v, ci_v, cb_v, ring_v,
                      ssem, rsem):
        wid = lax.axis_index("s") * _NUM_CORES + lax.axis_index("c")
        slab_lo = wid * (_NCHUNK * _CHUNK)
        iota = lax.iota(jnp.int32, 16)
        jrow = iota
        jhi = iota + 16

        def chunk_col(c):
            nom = wid * (_NCHUNK * _CHUNK // _LANES) + c * (_CHUNK // _LANES)
            return jnp.minimum(nom, ncols - _CHUNK // _LANES)

        def issue_chunk(tabT_hbm, c, par):
            start = pl.multiple_of(chunk_col(c) * _LANES, _LANES)
            pltpu.async_copy(tabT_hbm.at[:, pl.ds(start, _CHUNK)],
                             stage_v.at[par], ssem.at[par])

        def wait_chunk(tabT_hbm, par):
            pltpu.make_async_copy(tabT_hbm.at[:, pl.ds(0, _CHUNK)],
                                  stage_v.at[par], ssem.at[par]).wait()

        def wait_ring(out_hbm, l):
            pltpu.make_async_copy(out_hbm.at[pl.ds(0, D)],
                                  ring_v.at[l], rsem.at[l]).wait()

        def run_table(idx_hbm, tabT_hbm, out_hbm):
            issue_chunk(tabT_hbm, 0, 0)
            issue_chunk(tabT_hbm, 1, 1)
            pad_off = B * D + wid * D
            for l in range(_RING):
                pltpu.async_copy(ring_v.at[l], out_hbm.at[pl.ds(pad_off, D)],
                                 rsem.at[l])
            pltpu.sync_copy(idx_hbm, idx_v)

            def scan(k, off):
                v = idx_v[pl.ds(k * 16, 16)]
                m = (v >= slab_lo) & (v < slab_lo + _NCHUNK * _CHUNK)
                b = k * 16 + iota
                offc = jnp.minimum(off, _HIT_CAP)
                plsc.store_compressed(hi_v.at[pl.ds(offc, 16)], v, mask=m)
                plsc.store_compressed(hb_v.at[pl.ds(offc, 16)], b, mask=m)
                return off + plsc.all_reduce_population_count(m)[0]

            cnt = jnp.minimum(lax.fori_loop(0, B // 16, scan, 0), _HIT_CAP)


            def do_chunk(c, par):
                wait_chunk(tabT_hbm, par)
                base = chunk_col(c) * _LANES
                nom_lo = slab_lo + c * _CHUNK

                def cscan(k, off):
                    v = hi_v[pl.ds(k * 16, 16)]
                    b = hb_v[pl.ds(k * 16, 16)]
                    m = (v >= nom_lo) & (v < nom_lo + _CHUNK)
                    offc = jnp.minimum(off, _CHIT_CAP)
                    plsc.store_compressed(ci_v.at[pl.ds(offc, 16)], v, mask=m)
                    plsc.store_compressed(cb_v.at[pl.ds(offc, 16)], b, mask=m)
                    return off + plsc.all_reduce_population_count(m)[0]

                nvec = (cnt + 15) >> 4
                ccnt = jnp.minimum(lax.fori_loop(0, nvec, cscan, 0),
                                   _CHIT_CAP)
                bfill = jnp.broadcast_to(B + wid, (16,))
                ci_v[pl.ds(ccnt, 16)] = jnp.broadcast_to(base, (16,))
                cb_v[pl.ds(ccnt, 16)] = bfill
                ci_v[pl.ds(ccnt + 16, 16)] = jnp.broadcast_to(base, (16,))
                cb_v[pl.ds(ccnt + 16, 16)] = bfill

                def group(g, _):
                    for half in range(2):
                        civ = ci_v[pl.ds(g * 32 + half * 16, 16)] - base
                        cbv = cb_v[pl.ds(g * 32 + half * 16, 16)]
                        for l in range(16):
                            slot = (half * 16 + l) % _RING
                            wait_ring(out_hbm, slot)
                            col = jnp.broadcast_to(civ[l], (16,))
                            g0 = plsc.load_gather(stage_v.at[par],
                                                  [jrow, col])
                            g1 = plsc.load_gather(stage_v.at[par],
                                                  [jhi, col])
                            ring_v[slot, pl.ds(0, 16)] = g0
                            ring_v[slot, pl.ds(16, 16)] = g1
                            b = cbv[l]
                            pltpu.async_copy(
                                ring_v.at[slot], out_hbm.at[pl.ds(b * D, D)],
                                rsem.at[slot])
                    return _

                lax.fori_loop(0, (ccnt + 32) >> 5, group, 0)
                issue_chunk(tabT_hbm, c + 2, par)

            def pair(cp, _):
                do_chunk(cp * 2, 0)
                do_chunk(cp * 2 + 1, 1)
                return _

            lax.fori_loop(0, _NCHUNK // 2, pair, 0)
            wait_chunk(tabT_hbm, 0)
            wait_chunk(tabT_hbm, 1)
            for l in range(_RING):
                wait_ring(out_hbm, l)

        run_table(uidx_hbm, utT_hbm, uout_hbm)
        run_table(iidx_hbm, itT_hbm, iout_hbm)

    return gather_kernel


def _mlp_body(u_ref, v_ref, w1a_ref, w1b_ref, b1_ref, w2_ref, b2_ref,
              w3_ref, b3_ref, w4_ref, b4_ref, out_ref):
    dot = functools.partial(jnp.dot, preferred_element_type=jnp.float32,
                            precision=lax.Precision.HIGHEST)
    x = dot(u_ref[...], w1a_ref[...]) + dot(v_ref[...], w1b_ref[...])
    x = jnp.maximum(x + b1_ref[...], 0.0)
    x = jnp.maximum(dot(x, w2_ref[...]) + b2_ref[...], 0.0)
    x = jnp.maximum(dot(x, w3_ref[...]) + b3_ref[...], 0.0)
    logits = jnp.sum(x * w4_ref[...], axis=1) + b4_ref[0]
    out_ref[...] = jax.nn.sigmoid(logits)


def _make_mlp(B, D, blk):
    grid = (B // blk,)
    full = lambda shape: pl.BlockSpec(shape, lambda i: (0,) * len(shape))
    return pl.pallas_call(
        _mlp_body,
        grid=grid,
        in_specs=[
            pl.BlockSpec((blk, D), lambda i: (i, 0)),
            pl.BlockSpec((blk, D), lambda i: (i, 0)),
            full((D, 64)), full((D, 64)), full((1, 64)),
            full((64, 32)), full((1, 32)),
            full((32, 16)), full((1, 16)),
            full((1, 16)), full((1,)),
        ],
        out_specs=pl.BlockSpec((blk,), lambda i: (i,)),
        out_shape=jax.ShapeDtypeStruct((B,), jnp.float32),
    )


def kernel(user_indices, item_indices, user_table, item_table,
           W1, b1, W2, b2, W3, b3, W4, b4):
    B = user_indices.shape[0]
    D = user_table.shape[1]
    T = user_table.shape[0]
    uflat, iflat = _make_gather(B, D, T)(
        user_indices, item_indices, user_table.T, item_table.T)
    user_vec = uflat.reshape(B + _NUM_WORKERS, D)
    item_vec = iflat.reshape(B + _NUM_WORKERS, D)
    w1a = W1[:, :D].T
    w1b = W1[:, D:].T
    mlp = _make_mlp(B, D, 2048)
    return mlp(user_vec, item_vec,
               w1a, w1b, b1.reshape(1, 64),
               W2.T, b2.reshape(1, 32),
               W3.T, b3.reshape(1, 16),
               W4.reshape(1, 16), b4)


# speedup vs baseline: 1.1539x; 1.1488x over previous
import functools

import jax
import jax.numpy as jnp
from jax import lax
from jax.experimental import pallas as pl
from jax.experimental.pallas import tpu as pltpu
from jax.experimental.pallas import tpu_sc as plsc

_NUM_CORES = 2
_NUM_SUBCORES = 16
_NUM_WORKERS = _NUM_CORES * _NUM_SUBCORES
_LANES = 128
_CHUNK = 1024
_NCHUNK = 32
_HIT_CAP = 4096
_CHIT_CAP = 512
_RING = 16


def _make_gather(B, D, T):
    ncols = (T + _LANES - 1) // _LANES
    mesh = plsc.VectorSubcoreMesh(core_axis_name="c", subcore_axis_name="s")

    @functools.partial(
        pl.kernel,
        mesh=mesh,
        out_type=(
            jax.ShapeDtypeStruct((B * D + _NUM_WORKERS * D,), jnp.float32),
            jax.ShapeDtypeStruct((B * D + _NUM_WORKERS * D,), jnp.float32),
        ),
        scratch_types=[
            pltpu.VMEM((B,), jnp.int32),
            pltpu.VMEM((2, D, _CHUNK), jnp.float32),
            pltpu.VMEM((_HIT_CAP + 16,), jnp.int32),
            pltpu.VMEM((_HIT_CAP + 16,), jnp.int32),
            pltpu.VMEM((_CHIT_CAP + 32,), jnp.int32),
            pltpu.VMEM((_CHIT_CAP + 32,), jnp.int32),
            pltpu.VMEM((_RING, D), jnp.float32),
            pltpu.SemaphoreType.DMA((2,)),
            pltpu.SemaphoreType.DMA((_RING,)),
        ],
        compiler_params=pltpu.CompilerParams(use_tc_tiling_on_sc=True,
                                             needs_layout_passes=False),
    )
    def gather_kernel(uidx_hbm, iidx_hbm, utT_hbm, itT_hbm,
                      uout_hbm, iout_hbm,
                      idx_v, stage_v, hi_v, hb_v, ci_v, cb_v, ring_v,
                      ssem, rsem):
        wid = lax.axis_index("s") * _NUM_CORES + lax.axis_index("c")
        slab_lo = wid * (_NCHUNK * _CHUNK)
        iota = lax.iota(jnp.int32, 16)
        jrow = iota
        jhi = iota + 16

        def chunk_col(c):
            nom = wid * (_NCHUNK * _CHUNK // _LANES) + c * (_CHUNK // _LANES)
            return jnp.minimum(nom, ncols - _CHUNK // _LANES)

        def issue_chunk(tabT_hbm, c, par):
            start = pl.multiple_of(chunk_col(c) * _LANES, _LANES)
            pltpu.async_copy(tabT_hbm.at[:, pl.ds(start, _CHUNK)],
                             stage_v.at[par], ssem.at[par])

        def wait_chunk(tabT_hbm, par):
            pltpu.make_async_copy(tabT_hbm.at[:, pl.ds(0, _CHUNK)],
                                  stage_v.at[par], ssem.at[par]).wait()

        def wait_ring(out_hbm, l):
            pltpu.make_async_copy(out_hbm.at[pl.ds(0, D)],
                                  ring_v.at[l], rsem.at[l]).wait()

        def run_table(idx_hbm, tabT_hbm, out_hbm):
            issue_chunk(tabT_hbm, 0, 0)
            issue_chunk(tabT_hbm, 1, 1)
            pad_off = B * D + wid * D
            for l in range(_RING):
                pltpu.async_copy(ring_v.at[l], out_hbm.at[pl.ds(pad_off, D)],
                                 rsem.at[l])
            pltpu.sync_copy(idx_hbm, idx_v)

            def scan(k, off):
                v = idx_v[pl.ds(k * 16, 16)]
                m = (v >= slab_lo) & (v < slab_lo + _NCHUNK * _CHUNK)
                b = k * 16 + iota
                offc = jnp.minimum(off, _HIT_CAP)
                plsc.store_compressed(hi_v.at[pl.ds(offc, 16)], v, mask=m)
                plsc.store_compressed(hb_v.at[pl.ds(offc, 16)], b, mask=m)
                return off + plsc.all_reduce_population_count(m)[0]

            cnt = jnp.minimum(lax.fori_loop(0, B // 16, scan, 0), _HIT_CAP)


            def do_chunk(c, par):
                wait_chunk(tabT_hbm, par)
                base = chunk_col(c) * _LANES
                nom_lo = slab_lo + c * _CHUNK

                def cscan(k, off):
                    v = hi_v[pl.ds(k * 16, 16)]
                    b = hb_v[pl.ds(k * 16, 16)]
                    m = (v >= nom_lo) & (v < nom_lo + _CHUNK)
                    offc = jnp.minimum(off, _CHIT_CAP)
                    plsc.store_compressed(ci_v.at[pl.ds(offc, 16)], v, mask=m)
                    plsc.store_compressed(cb_v.at[pl.ds(offc, 16)], b, mask=m)
                    return off + plsc.all_reduce_population_count(m)[0]

                nvec = (cnt + 15) >> 4
                ccnt = jnp.minimum(lax.fori_loop(0, nvec, cscan, 0),
                                   _CHIT_CAP)
                bfill = jnp.broadcast_to(B + wid, (16,))
                ci_v[pl.ds(ccnt, 16)] = jnp.broadcast_to(base, (16,))
                cb_v[pl.ds(ccnt, 16)] = bfill
                ci_v[pl.ds(ccnt + 16, 16)] = jnp.broadcast_to(base, (16,))
                cb_v[pl.ds(ccnt + 16, 16)] = bfill

                def group(g, _):
                    for half in range(2):
                        civ = ci_v[pl.ds(g * 32 + half * 16, 16)] - base
                        cbv = cb_v[pl.ds(g * 32 + half * 16, 16)]
                        for l in range(16):
                            slot = (half * 16 + l) % _RING
                            wait_ring(out_hbm, slot)
                            col = jnp.broadcast_to(civ[l], (16,))
                            g0 = plsc.load_gather(stage_v.at[par],
                                                  [jrow, col])
                            g1 = plsc.load_gather(stage_v.at[par],
                                                  [jhi, col])
                            ring_v[slot, pl.ds(0, 16)] = g0
                            ring_v[slot, pl.ds(16, 16)] = g1
                            b = cbv[l]
                            pltpu.async_copy(
                                ring_v.at[slot], out_hbm.at[pl.ds(b * D, D)],
                                rsem.at[slot])
                    return _

                lax.fori_loop(0, (ccnt + 32) >> 5, group, 0)
                issue_chunk(tabT_hbm, c + 2, par)

            def pair(cp, _):
                do_chunk(cp * 2, 0)
                do_chunk(cp * 2 + 1, 1)
                return _

            lax.fori_loop(0, _NCHUNK // 2, pair, 0)
            wait_chunk(tabT_hbm, 0)
            wait_chunk(tabT_hbm, 1)
            for l in range(_RING):
                wait_ring(out_hbm, l)

        run_table(uidx_hbm, utT_hbm, uout_hbm)
        run_table(iidx_hbm, itT_hbm, iout_hbm)

    return gather_kernel


def _mlp_body(u_ref, v_ref, w1a_ref, w1b_ref, b1_ref, w2_ref, b2_ref,
              w3_ref, b3_ref, w4_ref, b4_ref, out_ref):
    dot = functools.partial(jnp.dot, preferred_element_type=jnp.float32)
    x = dot(u_ref[...], w1a_ref[...]) + dot(v_ref[...], w1b_ref[...])
    x = jnp.maximum(x + b1_ref[...], 0.0)
    x = jnp.maximum(dot(x, w2_ref[...]) + b2_ref[...], 0.0)
    x = jnp.maximum(dot(x, w3_ref[...]) + b3_ref[...], 0.0)
    logits = jnp.sum(x * w4_ref[...], axis=1) + b4_ref[0]
    out_ref[...] = jax.nn.sigmoid(logits)


def _make_mlp(B, D, blk):
    grid = (B // blk,)
    full = lambda shape: pl.BlockSpec(shape, lambda i: (0,) * len(shape))
    return pl.pallas_call(
        _mlp_body,
        grid=grid,
        in_specs=[
            pl.BlockSpec((blk, D), lambda i: (i, 0)),
            pl.BlockSpec((blk, D), lambda i: (i, 0)),
            full((D, 64)), full((D, 64)), full((1, 64)),
            full((64, 32)), full((1, 32)),
            full((32, 16)), full((1, 16)),
            full((1, 16)), full((1,)),
        ],
        out_specs=pl.BlockSpec((blk,), lambda i: (i,)),
        out_shape=jax.ShapeDtypeStruct((B,), jnp.float32),
    )


def kernel(user_indices, item_indices, user_table, item_table,
           W1, b1, W2, b2, W3, b3, W4, b4):
    B = user_indices.shape[0]
    D = user_table.shape[1]
    T = user_table.shape[0]
    uflat, iflat = _make_gather(B, D, T)(
        user_indices, item_indices, user_table.T, item_table.T)
    user_vec = uflat.reshape(B + _NUM_WORKERS, D)
    item_vec = iflat.reshape(B + _NUM_WORKERS, D)
    w1a = W1[:, :D].T
    w1b = W1[:, D:].T
    mlp = _make_mlp(B, D, 2048)
    return mlp(user_vec, item_vec,
               w1a, w1b, b1.reshape(1, 64),
               W2.T, b2.reshape(1, 32),
               W3.T, b3.reshape(1, 16),
               W4.reshape(1, 16), b4)


# speedup vs baseline: 1.3474x; 1.1677x over previous
import functools

import jax
import jax.numpy as jnp
from jax import lax
from jax.experimental import pallas as pl
from jax.experimental.pallas import tpu as pltpu
from jax.experimental.pallas import tpu_sc as plsc

_NUM_CORES = 2
_NUM_SUBCORES = 16
_NUM_WORKERS = _NUM_CORES * _NUM_SUBCORES
_LANES = 128
_CHUNK = 1024
_NCHUNK = 32
_HIT_CAP = 4096
_CHIT_CAP = 512
_RING = 16


def _make_gather(B, D, T):
    ncols = (T + _LANES - 1) // _LANES
    mesh = plsc.VectorSubcoreMesh(core_axis_name="c", subcore_axis_name="s")

    @functools.partial(
        pl.kernel,
        mesh=mesh,
        out_type=(
            jax.ShapeDtypeStruct((B * D + _NUM_WORKERS * D,), jnp.float32),
            jax.ShapeDtypeStruct((B * D + _NUM_WORKERS * D,), jnp.float32),
        ),
        scratch_types=[
            pltpu.VMEM((B,), jnp.int32),
            pltpu.VMEM((2, D, _CHUNK), jnp.float32),
            pltpu.VMEM((_HIT_CAP + 16,), jnp.int32),
            pltpu.VMEM((_HIT_CAP + 16,), jnp.int32),
            pltpu.VMEM((_CHIT_CAP + 32,), jnp.int32),
            pltpu.VMEM((_CHIT_CAP + 32,), jnp.int32),
            pltpu.VMEM((_RING, D), jnp.float32),
            pltpu.SemaphoreType.DMA((2,)),
            pltpu.SemaphoreType.DMA((_RING,)),
        ],
        compiler_params=pltpu.CompilerParams(use_tc_tiling_on_sc=True,
                                             needs_layout_passes=False),
    )
    def gather_kernel(uidx_hbm, iidx_hbm, utT_hbm, itT_hbm,
                      uout_hbm, iout_hbm,
                      idx_v, stage_v, hi_v, hb_v, ci_v, cb_v, ring_v,
                      ssem, rsem):
        wid = lax.axis_index("s") * _NUM_CORES + lax.axis_index("c")
        slab_lo = wid * (_NCHUNK * _CHUNK)
        iota = lax.iota(jnp.int32, 16)
        jrow = iota
        jhi = iota + 16

        def chunk_col(c):
            nom = wid * (_NCHUNK * _CHUNK // _LANES) + c * (_CHUNK // _LANES)
            return jnp.minimum(nom, ncols - _CHUNK // _LANES)

        def issue_chunk(tabT_hbm, c, par):
            start = pl.multiple_of(chunk_col(c) * _LANES, _LANES)
            pltpu.async_copy(tabT_hbm.at[:, pl.ds(start, _CHUNK)],
                             stage_v.at[par], ssem.at[par])

        def wait_chunk(tabT_hbm, par):
            pltpu.make_async_copy(tabT_hbm.at[:, pl.ds(0, _CHUNK)],
                                  stage_v.at[par], ssem.at[par]).wait()

        def wait_ring(out_hbm, l):
            pltpu.make_async_copy(out_hbm.at[pl.ds(0, D)],
                                  ring_v.at[l], rsem.at[l]).wait()

        def run_table(idx_hbm, tabT_hbm, out_hbm):
            issue_chunk(tabT_hbm, 0, 0)
            issue_chunk(tabT_hbm, 1, 1)
            pad_off = B * D + wid * D
            for l in range(_RING):
                pltpu.async_copy(ring_v.at[l], out_hbm.at[pl.ds(pad_off, D)],
                                 rsem.at[l])
            pltpu.sync_copy(idx_hbm, idx_v)

            def scan(k, off):
                v = idx_v[pl.ds(k * 16, 16)]
                m = (v >= slab_lo) & (v < slab_lo + _NCHUNK * _CHUNK)
                b = k * 16 + iota
                offc = jnp.minimum(off, _HIT_CAP)
                plsc.store_compressed(hi_v.at[pl.ds(offc, 16)], v, mask=m)
                plsc.store_compressed(hb_v.at[pl.ds(offc, 16)], b, mask=m)
                return off + plsc.all_reduce_population_count(m)[0]

            cnt = jnp.minimum(lax.fori_loop(0, B // 16, scan, 0), _HIT_CAP)


            def do_chunk(c, par):
                wait_chunk(tabT_hbm, par)
                base = chunk_col(c) * _LANES
                nom_lo = slab_lo + c * _CHUNK

                def cscan(k, off):
                    v = hi_v[pl.ds(k * 16, 16)]
                    b = hb_v[pl.ds(k * 16, 16)]
                    m = (v >= nom_lo) & (v < nom_lo + _CHUNK)
                    offc = jnp.minimum(off, _CHIT_CAP)
                    plsc.store_compressed(ci_v.at[pl.ds(offc, 16)], v, mask=m)
                    plsc.store_compressed(cb_v.at[pl.ds(offc, 16)], b, mask=m)
                    return off + plsc.all_reduce_population_count(m)[0]

                nvec = (cnt + 15) >> 4
                ccnt = jnp.minimum(lax.fori_loop(0, nvec, cscan, 0),
                                   _CHIT_CAP)
                bfill = jnp.broadcast_to(B + wid, (16,))
                ci_v[pl.ds(ccnt, 16)] = jnp.broadcast_to(base, (16,))
                cb_v[pl.ds(ccnt, 16)] = bfill

                def group(g, _):
                    civ = ci_v[pl.ds(g * 16, 16)] - base
                    cbv = cb_v[pl.ds(g * 16, 16)]
                    for l in range(16):
                        wait_ring(out_hbm, l)
                        col = jnp.broadcast_to(civ[l], (16,))
                        g0 = plsc.load_gather(stage_v.at[par], [jrow, col])
                        g1 = plsc.load_gather(stage_v.at[par], [jhi, col])
                        ring_v[l, pl.ds(0, 16)] = g0
                        ring_v[l, pl.ds(16, 16)] = g1
                        b = cbv[l]
                        pltpu.async_copy(
                            ring_v.at[l], out_hbm.at[pl.ds(b * D, D)],
                            rsem.at[l])
                    return _

                lax.fori_loop(0, (ccnt + 16) >> 4, group, 0)
                issue_chunk(tabT_hbm, c + 2, par)

            def pair(cp, _):
                do_chunk(cp * 2, 0)
                do_chunk(cp * 2 + 1, 1)
                return _

            lax.fori_loop(0, _NCHUNK // 2, pair, 0)
            wait_chunk(tabT_hbm, 0)
            wait_chunk(tabT_hbm, 1)
            for l in range(_RING):
                wait_ring(out_hbm, l)

        run_table(uidx_hbm, utT_hbm, uout_hbm)
        run_table(iidx_hbm, itT_hbm, iout_hbm)

    return gather_kernel


def _mlp_body(u_ref, v_ref, w1a_ref, w1b_ref, b1_ref, w2_ref, b2_ref,
              w3_ref, b3_ref, w4_ref, b4_ref, out_ref):
    dot = functools.partial(jnp.dot, preferred_element_type=jnp.float32)
    bf = jnp.bfloat16
    x = (dot(u_ref[...].astype(bf), w1a_ref[...].astype(bf))
         + dot(v_ref[...].astype(bf), w1b_ref[...].astype(bf)))
    x = jnp.maximum(x + b1_ref[...], 0.0)
    x = jnp.maximum(dot(x.astype(bf), w2_ref[...].astype(bf))
                    + b2_ref[...], 0.0)
    x = jnp.maximum(dot(x.astype(bf), w3_ref[...].astype(bf))
                    + b3_ref[...], 0.0)
    logits = jnp.sum(x * w4_ref[...], axis=1) + b4_ref[0]
    out_ref[...] = jax.nn.sigmoid(logits)


def _make_mlp(B, D, blk):
    grid = (B // blk,)
    full = lambda shape: pl.BlockSpec(shape, lambda i: (0,) * len(shape))
    return pl.pallas_call(
        _mlp_body,
        grid=grid,
        in_specs=[
            pl.BlockSpec((blk, D), lambda i: (i, 0)),
            pl.BlockSpec((blk, D), lambda i: (i, 0)),
            full((D, 64)), full((D, 64)), full((1, 64)),
            full((64, 32)), full((1, 32)),
            full((32, 16)), full((1, 16)),
            full((1, 16)), full((1,)),
        ],
        out_specs=pl.BlockSpec((blk,), lambda i: (i,)),
        out_shape=jax.ShapeDtypeStruct((B,), jnp.float32),
    )


def kernel(user_indices, item_indices, user_table, item_table,
           W1, b1, W2, b2, W3, b3, W4, b4):
    B = user_indices.shape[0]
    D = user_table.shape[1]
    T = user_table.shape[0]
    uflat, iflat = _make_gather(B, D, T)(
        user_indices, item_indices, user_table.T, item_table.T)
    user_vec = uflat.reshape(B + _NUM_WORKERS, D)
    item_vec = iflat.reshape(B + _NUM_WORKERS, D)
    w1a = W1[:, :D].T
    w1b = W1[:, D:].T
    mlp = _make_mlp(B, D, 2048)
    return mlp(user_vec, item_vec,
               w1a, w1b, b1.reshape(1, 64),
               W2.T, b2.reshape(1, 32),
               W3.T, b3.reshape(1, 16),
               W4.reshape(1, 16), b4)


# speedup vs baseline: 1.3828x; 1.0263x over previous
import functools

import jax
import jax.numpy as jnp
from jax import lax
from jax.experimental import pallas as pl
from jax.experimental.pallas import tpu as pltpu
from jax.experimental.pallas import tpu_sc as plsc

_NUM_CORES = 2
_NUM_SUBCORES = 16
_NUM_WORKERS = _NUM_CORES * _NUM_SUBCORES
_LANES = 128
_CHUNK = 1024
_NCHUNK = 32
_HIT_CAP = 4096
_CHIT_CAP = 512
_RING = 16


def _make_gather(B, D, T):
    ncols = (T + _LANES - 1) // _LANES
    mesh = plsc.VectorSubcoreMesh(core_axis_name="c", subcore_axis_name="s")

    @functools.partial(
        pl.kernel,
        mesh=mesh,
        out_type=(
            jax.ShapeDtypeStruct((B * D + _NUM_WORKERS * D,), jnp.float32),
            jax.ShapeDtypeStruct((B * D + _NUM_WORKERS * D,), jnp.float32),
        ),
        scratch_types=[
            pltpu.VMEM((B,), jnp.int32),
            pltpu.VMEM((2, D, _CHUNK), jnp.float32),
            pltpu.VMEM((_HIT_CAP + 16,), jnp.int32),
            pltpu.VMEM((_HIT_CAP + 16,), jnp.int32),
            pltpu.VMEM((_CHIT_CAP + 32,), jnp.int32),
            pltpu.VMEM((_CHIT_CAP + 32,), jnp.int32),
            pltpu.VMEM((_RING, D), jnp.float32),
            pltpu.SemaphoreType.DMA((2,)),
            pltpu.SemaphoreType.DMA((_RING,)),
        ],
        compiler_params=pltpu.CompilerParams(use_tc_tiling_on_sc=True,
                                             needs_layout_passes=False),
    )
    def gather_kernel(uidx_hbm, iidx_hbm, utT_hbm, itT_hbm,
                      uout_hbm, iout_hbm,
                      idx_v, stage_v, hi_v, hb_v, ci_v, cb_v, ring_v,
                      ssem, rsem):
        wid = lax.axis_index("s") * _NUM_CORES + lax.axis_index("c")
        slab_lo = wid * (_NCHUNK * _CHUNK)
        iota = lax.iota(jnp.int32, 16)
        jrow = iota
        jhi = iota + 16

        def chunk_col(c):
            nom = wid * (_NCHUNK * _CHUNK // _LANES) + c * (_CHUNK // _LANES)
            return jnp.minimum(nom, ncols - _CHUNK // _LANES)

        def issue_chunk(tabT_hbm, c, par):
            start = pl.multiple_of(chunk_col(c) * _LANES, _LANES)
            pltpu.async_copy(tabT_hbm.at[:, pl.ds(start, _CHUNK)],
                             stage_v.at[par], ssem.at[par])

        def wait_chunk(tabT_hbm, par):
            pltpu.make_async_copy(tabT_hbm.at[:, pl.ds(0, _CHUNK)],
                                  stage_v.at[par], ssem.at[par]).wait()

        def wait_ring(out_hbm, l):
            pltpu.make_async_copy(out_hbm.at[pl.ds(0, D)],
                                  ring_v.at[l], rsem.at[l]).wait()

        def run_table(idx_hbm, tabT_hbm, out_hbm):
            issue_chunk(tabT_hbm, 0, 0)
            issue_chunk(tabT_hbm, 1, 1)
            pad_off = B * D + wid * D
            for l in range(_RING):
                pltpu.async_copy(ring_v.at[l], out_hbm.at[pl.ds(pad_off, D)],
                                 rsem.at[l])
            pltpu.sync_copy(idx_hbm, idx_v)

            def scan(k, off):
                v = idx_v[pl.ds(k * 16, 16)]
                m = (v >= slab_lo) & (v < slab_lo + _NCHUNK * _CHUNK)
                b = k * 16 + iota
                offc = jnp.minimum(off, _HIT_CAP)
                plsc.store_compressed(hi_v.at[pl.ds(offc, 16)], v, mask=m)
                plsc.store_compressed(hb_v.at[pl.ds(offc, 16)], b, mask=m)
                return off + plsc.all_reduce_population_count(m)[0]

            cnt = jnp.minimum(lax.fori_loop(0, B // 16, scan, 0), _HIT_CAP)


            def do_chunk(c, par):
                wait_chunk(tabT_hbm, par)
                base = chunk_col(c) * _LANES
                nom_lo = slab_lo + c * _CHUNK

                def cscan(k, off):
                    v = hi_v[pl.ds(k * 16, 16)]
                    b = hb_v[pl.ds(k * 16, 16)]
                    m = (v >= nom_lo) & (v < nom_lo + _CHUNK)
                    offc = jnp.minimum(off, _CHIT_CAP)
                    plsc.store_compressed(ci_v.at[pl.ds(offc, 16)], v, mask=m)
                    plsc.store_compressed(cb_v.at[pl.ds(offc, 16)], b, mask=m)
                    return off + plsc.all_reduce_population_count(m)[0]

                nvec = (cnt + 15) >> 4
                ccnt = jnp.minimum(lax.fori_loop(0, nvec, cscan, 0),
                                   _CHIT_CAP)
                bfill = jnp.broadcast_to(B + wid, (16,))
                ci_v[pl.ds(ccnt, 16)] = jnp.broadcast_to(base, (16,))
                cb_v[pl.ds(ccnt, 16)] = bfill

                def group(g, _):
                    civ = ci_v[pl.ds(g * 16, 16)] - base
                    cbv = cb_v[pl.ds(g * 16, 16)]
                    for l in range(16):
                        wait_ring(out_hbm, l)
                        col = jnp.broadcast_to(civ[l], (16,))
                        g0 = plsc.load_gather(stage_v.at[par], [jrow, col])
                        g1 = plsc.load_gather(stage_v.at[par], [jhi, col])
                        ring_v[l, pl.ds(0, 16)] = g0
                        ring_v[l, pl.ds(16, 16)] = g1
                        b = cbv[l]
                        pltpu.async_copy(
                            ring_v.at[l], out_hbm.at[pl.ds(b * D, D)],
                            rsem.at[l])
                    return _

                lax.fori_loop(0, (ccnt + 16) >> 4, group, 0)
                issue_chunk(tabT_hbm, c + 2, par)

            def pair(cp, _):
                do_chunk(cp * 2, 0)
                do_chunk(cp * 2 + 1, 1)
                return _

            lax.fori_loop(0, _NCHUNK // 2, pair, 0)
            wait_chunk(tabT_hbm, 0)
            wait_chunk(tabT_hbm, 1)
            for l in range(_RING):
                wait_ring(out_hbm, l)

        run_table(uidx_hbm, utT_hbm, uout_hbm)
        run_table(iidx_hbm, itT_hbm, iout_hbm)

    return gather_kernel


def _mlp_body(u_ref, v_ref, w1a_ref, w1b_ref, b1_ref, w2_ref, b2_ref,
              w3_ref, b3_ref, w4_ref, b4_ref, out_ref):
    dot = functools.partial(jnp.dot, preferred_element_type=jnp.float32)
    bf = jnp.bfloat16
    x = (dot(u_ref[...].astype(bf), w1a_ref[...].astype(bf))
         + dot(v_ref[...].astype(bf), w1b_ref[...].astype(bf)))
    x = jnp.maximum(x + b1_ref[...], 0.0)
    x = jnp.maximum(dot(x.astype(bf), w2_ref[...].astype(bf))
                    + b2_ref[...], 0.0)
    x = jnp.maximum(dot(x.astype(bf), w3_ref[...].astype(bf))
                    + b3_ref[...], 0.0)
    logits = jnp.sum(x * w4_ref[...], axis=1) + b4_ref[0]
    out_ref[...] = jax.nn.sigmoid(logits)


def _make_mlp(B, D, blk):
    grid = (B // blk,)
    full = lambda shape: pl.BlockSpec(shape, lambda i: (0,) * len(shape))
    return pl.pallas_call(
        _mlp_body,
        grid=grid,
        in_specs=[
            pl.BlockSpec((blk, D), lambda i: (i, 0)),
            pl.BlockSpec((blk, D), lambda i: (i, 0)),
            full((D, 64)), full((D, 64)), full((1, 64)),
            full((64, 32)), full((1, 32)),
            full((32, 16)), full((1, 16)),
            full((1, 16)), full((1,)),
        ],
        out_specs=pl.BlockSpec((blk,), lambda i: (i,)),
        out_shape=jax.ShapeDtypeStruct((B,), jnp.float32),
    )


def kernel(user_indices, item_indices, user_table, item_table,
           W1, b1, W2, b2, W3, b3, W4, b4):
    B = user_indices.shape[0]
    D = user_table.shape[1]
    T = user_table.shape[0]
    uflat, iflat = _make_gather(B, D, T)(
        user_indices, item_indices, user_table.T, item_table.T)
    user_vec = uflat.reshape(B + _NUM_WORKERS, D)
    item_vec = iflat.reshape(B + _NUM_WORKERS, D)
    w1a = W1[:, :D].T
    w1b = W1[:, D:].T
    mlp = _make_mlp(B, D, 4096)
    return mlp(user_vec, item_vec,
               w1a, w1b, b1.reshape(1, 64),
               W2.T, b2.reshape(1, 32),
               W3.T, b3.reshape(1, 16),
               W4.reshape(1, 16), b4)
